# baseline (speedup 1.0000x reference)
import functools

import jax
import jax.numpy as jnp
from jax import lax
from jax.experimental import pallas as pl
from jax.experimental.pallas import tpu as pltpu
from jax.experimental.pallas import tpu_sc as plsc


def _make_sc_kernel(B, D, V, NC, NW, L):
    ROW_MAIN = (V // 128) * 128
    TAIL = V - ROW_MAIN
    F_PER_W = D // NW
    CHUNK = 2048
    n_chunks = B // CHUNK
    inv_n = 1.0 / (B * D)

    mesh = plsc.VectorSubcoreMesh(core_axis_name="c", subcore_axis_name="s")

    @functools.partial(
        pl.kernel,
        mesh=mesh,
        compiler_params=pltpu.CompilerParams(use_tc_tiling_on_sc=True,
                                             needs_layout_passes=False),
        out_type=jax.ShapeDtypeStruct((NW, 8, 128), jnp.float32),
        scratch_types=[
            pltpu.VMEM((ROW_MAIN + 128,), jnp.float32),
            pltpu.VMEM((B,), jnp.int32),
            pltpu.VMEM((CHUNK,), jnp.float32),
            pltpu.VMEM((CHUNK,), jnp.float32),
            pltpu.VMEM((8, 128), jnp.float32),
            pltpu.SemaphoreType.DMA,
            pltpu.SemaphoreType.DMA,
        ],
    )
    def sc_kernel(x_hbm, idx_hbm, tab_hbm, tail_hbm, out_hbm, row_v, lab_v,
                  xv0, xv1, ob_v, lsem, xsem):
        wid = lax.axis_index("s") * NC + lax.axis_index("c")
        xvs = (xv0, xv1)

        pltpu.async_copy(idx_hbm.at[pl.ds(0, B)], lab_v, lsem)

        def feat_body(fi, acc):
            f = F_PER_W * wid + fi
            t = lax.shift_right_logical(f, 3)
            r = lax.bitwise_and(f, 7)

            pltpu.sync_copy(tab_hbm.at[t, r, pl.ds(0, ROW_MAIN)],
                            row_v.at[pl.ds(0, ROW_MAIN)])
            if TAIL:
                pltpu.sync_copy(tail_hbm.at[t, r, pl.ds(0, 128)],
                                row_v.at[pl.ds(ROW_MAIN, 128)])

            def fire(c, buf):
                pltpu.async_copy(x_hbm.at[t, r, pl.ds(c * CHUNK, CHUNK)],
                                 xvs[buf], xsem)

            fire(0, 0)
            fire(1, 1)

            def pair_body(p, a):
                for buf in range(2):
                    pltpu.make_async_copy(
                        x_hbm.at[t, r, pl.ds(0, CHUNK)], xvs[buf],
                        xsem).wait()
                    base = (2 * p + buf) * CHUNK
                    xv = xvs[buf]

                    def body(g, aa):
                        cv = plsc.load_gather(
                            row_v, [lab_v[pl.ds(base + g * L, L)]])
                        d = xv[pl.ds(g * L, L)] - cv
                        return aa + d * d

                    a = lax.fori_loop(0, CHUNK // L, body, a)

                    @pl.when(p + 1 < n_chunks // 2)
                    def _():
                        fire(2 * p + 2 + buf, buf)
                return a

            return lax.fori_loop(0, n_chunks // 2, pair_body, acc)

        pltpu.make_async_copy(idx_hbm.at[pl.ds(0, B)], lab_v, lsem).wait()
        acc = lax.fori_loop(0, F_PER_W, feat_body,
                            jnp.zeros((L,), jnp.float32))

        for rr in range(8):
            for h in range(128 // L):
                ob_v[rr, pl.ds(h * L, L)] = jnp.zeros((L,), jnp.float32)
        ob_v[0, pl.ds(0, L)] = acc * inv_n
        pltpu.sync_copy(ob_v, out_hbm.at[wid])

    return sc_kernel


def kernel(x, labels, centers):
    B, D = x.shape
    V = centers.shape[0]
    info = plsc.get_sparse_core_info()
    NC, NS, L = info.num_cores, info.num_subcores, info.num_lanes
    NW = NC * NS

    sc_kernel = _make_sc_kernel(B, D, V, NC, NW, L)
    row_main = (V // 128) * 128
    tail = jnp.pad(centers[row_main:].T, ((0, 0), (0, 128 - (V - row_main))))
    partials = sc_kernel(
        x.T.reshape(D // 8, 8, B),
        labels.astype(jnp.int32),
        centers.T.reshape(D // 8, 8, V),
        tail.reshape(D // 8, 8, 128),
    )
    return jnp.sum(partials)


# speedup vs baseline: 1.0375x; 1.0375x over previous
import functools

import jax
import jax.numpy as jnp
from jax import lax
from jax.experimental import pallas as pl
from jax.experimental.pallas import tpu as pltpu
from jax.experimental.pallas import tpu_sc as plsc


def _make_sc_kernel(B, D, V, NC, NW, L):
    ROW_MAIN = (V // 128) * 128
    TAIL = V - ROW_MAIN
    F_PER_W = D // NW
    CHUNK = 2048
    n_chunks = B // CHUNK
    inv_n = 1.0 / (B * D)

    mesh = plsc.VectorSubcoreMesh(core_axis_name="c", subcore_axis_name="s")

    @functools.partial(
        pl.kernel,
        mesh=mesh,
        compiler_params=pltpu.CompilerParams(use_tc_tiling_on_sc=True,
                                             needs_layout_passes=False),
        out_type=jax.ShapeDtypeStruct((NW, 8, 128), jnp.float32),
        scratch_types=[
            pltpu.VMEM((ROW_MAIN + 128,), jnp.float32),
            pltpu.VMEM((B,), jnp.int32),
            pltpu.VMEM((CHUNK,), jnp.float32),
            pltpu.VMEM((CHUNK,), jnp.float32),
            pltpu.VMEM((8, 128), jnp.float32),
            pltpu.SemaphoreType.DMA,
            pltpu.SemaphoreType.DMA,
        ],
    )
    def sc_kernel(x_hbm, idx_hbm, tab_hbm, tail_hbm, out_hbm, row_v, lab_v,
                  xv0, xv1, ob_v, lsem, xsem):
        wid = lax.axis_index("s") * NC + lax.axis_index("c")
        xvs = (xv0, xv1)
        acc = jnp.zeros((L,), jnp.float32)

        pltpu.async_copy(idx_hbm.at[pl.ds(0, B)], lab_v, lsem)

        fs, ts, rs = [], [], []
        for fi in range(F_PER_W):
            f = F_PER_W * wid + fi
            fs.append(f)
            ts.append(lax.shift_right_logical(f, 3))
            rs.append(lax.bitwise_and(f, 7))

        def fire(fi, c, buf):
            pltpu.async_copy(
                x_hbm.at[ts[fi], rs[fi], pl.ds(c * CHUNK, CHUNK)],
                xvs[buf], xsem)

        fire(0, 0, 0)
        fire(0, 1, 1)

        for fi in range(F_PER_W):
            t, r = ts[fi], rs[fi]
            pltpu.sync_copy(tab_hbm.at[t, r, pl.ds(0, ROW_MAIN)],
                            row_v.at[pl.ds(0, ROW_MAIN)])
            if TAIL:
                pltpu.sync_copy(tail_hbm.at[t, r, pl.ds(0, 128)],
                                row_v.at[pl.ds(ROW_MAIN, 128)])
            if fi == 0:
                pltpu.make_async_copy(idx_hbm.at[pl.ds(0, B)], lab_v,
                                      lsem).wait()

            for c in range(n_chunks):
                buf = c % 2
                pltpu.make_async_copy(
                    x_hbm.at[t, r, pl.ds(0, CHUNK)], xvs[buf], xsem).wait()
                base = c * CHUNK
                xv = xvs[buf]

                def body(g, a):
                    cv = plsc.load_gather(
                        row_v, [lab_v[pl.ds(base + g * L, L)]])
                    d = xv[pl.ds(g * L, L)] - cv
                    return a + d * d

                acc = lax.fori_loop(0, CHUNK // L, body, acc)

                if c + 2 < n_chunks:
                    fire(fi, c + 2, buf)
                elif fi + 1 < F_PER_W:
                    fire(fi + 1, c + 2 - n_chunks, buf)

        for rr in range(8):
            for h in range(128 // L):
                ob_v[rr, pl.ds(h * L, L)] = jnp.zeros((L,), jnp.float32)
        ob_v[0, pl.ds(0, L)] = acc * inv_n
        pltpu.sync_copy(ob_v, out_hbm.at[wid])

    return sc_kernel


def kernel(x, labels, centers):
    B, D = x.shape
    V = centers.shape[0]
    info = plsc.get_sparse_core_info()
    NC, NS, L = info.num_cores, info.num_subcores, info.num_lanes
    NW = NC * NS

    sc_kernel = _make_sc_kernel(B, D, V, NC, NW, L)
    row_main = (V // 128) * 128
    tail = jnp.pad(centers[row_main:].T, ((0, 0), (0, 128 - (V - row_main))))
    partials = sc_kernel(
        x.T.reshape(D // 8, 8, B),
        labels.astype(jnp.int32),
        centers.T.reshape(D // 8, 8, V),
        tail.reshape(D // 8, 8, 128),
    )
    return jnp.sum(partials)


# speedup vs baseline: 1.1484x; 1.1069x over previous
import functools

import jax
import jax.numpy as jnp
from jax import lax
from jax.experimental import pallas as pl
from jax.experimental.pallas import tpu as pltpu
from jax.experimental.pallas import tpu_sc as plsc


def _make_sc_kernel(B, D, V, NC, NW, L):
    ROW_MAIN = (V // 128) * 128
    TAIL = V - ROW_MAIN
    F_PER_W = D // NW
    CHUNK = 2048
    n_chunks = B // CHUNK
    inv_n = 1.0 / (B * D)

    mesh = plsc.VectorSubcoreMesh(core_axis_name="c", subcore_axis_name="s")

    @functools.partial(
        pl.kernel,
        mesh=mesh,
        compiler_params=pltpu.CompilerParams(use_tc_tiling_on_sc=True,
                                             needs_layout_passes=False),
        out_type=jax.ShapeDtypeStruct((NW, 8, 128), jnp.float32),
        scratch_types=[
            pltpu.VMEM((ROW_MAIN + 128,), jnp.float32),
            pltpu.VMEM((B,), jnp.int32),
            pltpu.VMEM((CHUNK,), jnp.float32),
            pltpu.VMEM((CHUNK,), jnp.float32),
            pltpu.VMEM((8, 128), jnp.float32),
            pltpu.SemaphoreType.DMA,
            pltpu.SemaphoreType.DMA,
        ],
    )
    def sc_kernel(x_hbm, idx_hbm, tab_hbm, tail_hbm, out_hbm, row_v, lab_v,
                  xv0, xv1, ob_v, lsem, xsem):
        wid = lax.axis_index("s") * NC + lax.axis_index("c")
        xvs = (xv0, xv1)
        zero = jnp.zeros((L,), jnp.float32)
        acc = zero

        pltpu.async_copy(idx_hbm.at[pl.ds(0, B)], lab_v, lsem)

        fs, ts, rs = [], [], []
        for fi in range(F_PER_W):
            f = F_PER_W * wid + fi
            fs.append(f)
            ts.append(lax.shift_right_logical(f, 3))
            rs.append(lax.bitwise_and(f, 7))

        def fire(fi, c, buf):
            pltpu.async_copy(
                x_hbm.at[ts[fi], rs[fi], pl.ds(c * CHUNK, CHUNK)],
                xvs[buf], xsem)

        fire(0, 0, 0)
        fire(0, 1, 1)

        for fi in range(F_PER_W):
            t, r = ts[fi], rs[fi]
            pltpu.sync_copy(tab_hbm.at[t, r, pl.ds(0, ROW_MAIN)],
                            row_v.at[pl.ds(0, ROW_MAIN)])
            if TAIL:
                pltpu.sync_copy(tail_hbm.at[t, r, pl.ds(0, 128)],
                                row_v.at[pl.ds(ROW_MAIN, 128)])
            if fi == 0:
                pltpu.make_async_copy(idx_hbm.at[pl.ds(0, B)], lab_v,
                                      lsem).wait()

            for c in range(n_chunks):
                buf = c % 2
                pltpu.make_async_copy(
                    x_hbm.at[t, r, pl.ds(0, CHUNK)], xvs[buf], xsem).wait()
                base = c * CHUNK
                xv = xvs[buf]

                def body(blk, a):
                    out = []
                    for u in range(4):
                        off = base + (blk * 4 + u) * L
                        cv = plsc.load_gather(row_v, [lab_v[pl.ds(off, L)]])
                        d = xv[pl.ds((blk * 4 + u) * L, L)] - cv
                        out.append(a[u] + d * d)
                    return tuple(out)

                accs = lax.fori_loop(0, CHUNK // (4 * L), body,
                                     (acc, zero, zero, zero))
                acc = (accs[0] + accs[1]) + (accs[2] + accs[3])

                if c + 2 < n_chunks:
                    fire(fi, c + 2, buf)
                elif fi + 1 < F_PER_W:
                    fire(fi + 1, c + 2 - n_chunks, buf)

        for rr in range(8):
            for h in range(128 // L):
                ob_v[rr, pl.ds(h * L, L)] = jnp.zeros((L,), jnp.float32)
        ob_v[0, pl.ds(0, L)] = acc * inv_n
        pltpu.sync_copy(ob_v, out_hbm.at[wid])

    return sc_kernel


def kernel(x, labels, centers):
    B, D = x.shape
    V = centers.shape[0]
    info = plsc.get_sparse_core_info()
    NC, NS, L = info.num_cores, info.num_subcores, info.num_lanes
    NW = NC * NS

    sc_kernel = _make_sc_kernel(B, D, V, NC, NW, L)
    row_main = (V // 128) * 128
    tail = jnp.pad(centers[row_main:].T, ((0, 0), (0, 128 - (V - row_main))))
    partials = sc_kernel(
        x.T.reshape(D // 8, 8, B),
        labels.astype(jnp.int32),
        centers.T.reshape(D // 8, 8, V),
        tail.reshape(D // 8, 8, 128),
    )
    return jnp.sum(partials)


# speedup vs baseline: 1.1717x; 1.0202x over previous
import functools

import jax
import jax.numpy as jnp
from jax import lax
from jax.experimental import pallas as pl
from jax.experimental.pallas import tpu as pltpu
from jax.experimental.pallas import tpu_sc as plsc


def _make_sc_kernel(B, D, V, NC, NW, L):
    ROW_MAIN = (V // 128) * 128
    TAIL = V - ROW_MAIN
    F_PER_W = D // NW
    CHUNK = 2048
    n_chunks = B // CHUNK
    inv_n = 1.0 / (B * D)

    mesh = plsc.VectorSubcoreMesh(core_axis_name="c", subcore_axis_name="s")

    @functools.partial(
        pl.kernel,
        mesh=mesh,
        compiler_params=pltpu.CompilerParams(use_tc_tiling_on_sc=True,
                                             needs_layout_passes=False),
        out_type=jax.ShapeDtypeStruct((NW, 8, 128), jnp.float32),
        scratch_types=[
            pltpu.VMEM((ROW_MAIN + 128,), jnp.float32),
            pltpu.VMEM((B,), jnp.int32),
            pltpu.VMEM((CHUNK,), jnp.float32),
            pltpu.VMEM((CHUNK,), jnp.float32),
            pltpu.VMEM((8, 128), jnp.float32),
            pltpu.SemaphoreType.DMA,
            pltpu.SemaphoreType.DMA,
        ],
    )
    def sc_kernel(x_hbm, idx_hbm, tab_hbm, tail_hbm, out_hbm, row_v, lab_v,
                  xv0, xv1, ob_v, lsem, xsem):
        wid = lax.axis_index("s") * NC + lax.axis_index("c")
        xvs = (xv0, xv1)
        zero = jnp.zeros((L,), jnp.float32)
        acc = zero

        pltpu.async_copy(idx_hbm.at[pl.ds(0, B)], lab_v, lsem)

        fs, ts, rs = [], [], []
        for fi in range(F_PER_W):
            f = F_PER_W * wid + fi
            fs.append(f)
            ts.append(lax.shift_right_logical(f, 3))
            rs.append(lax.bitwise_and(f, 7))

        def fire(fi, c, buf):
            pltpu.async_copy(
                x_hbm.at[ts[fi], rs[fi], pl.ds(c * CHUNK, CHUNK)],
                xvs[buf], xsem)

        n_pairs = n_chunks // 2

        for fi in range(F_PER_W):
            t, r = ts[fi], rs[fi]
            fire(fi, 0, 0)
            fire(fi, 1, 1)
            pltpu.sync_copy(tab_hbm.at[t, r, pl.ds(0, ROW_MAIN)],
                            row_v.at[pl.ds(0, ROW_MAIN)])
            if TAIL:
                pltpu.sync_copy(tail_hbm.at[t, r, pl.ds(0, 128)],
                                row_v.at[pl.ds(ROW_MAIN, 128)])
            if fi == 0:
                pltpu.make_async_copy(idx_hbm.at[pl.ds(0, B)], lab_v,
                                      lsem).wait()

            def pair_body(p, a):
                for buf in range(2):
                    pltpu.make_async_copy(
                        x_hbm.at[t, r, pl.ds(0, CHUNK)], xvs[buf],
                        xsem).wait()
                    base = (2 * p + buf) * CHUNK
                    xv = xvs[buf]

                    def body(blk, aa):
                        out = []
                        for u in range(4):
                            off = base + (blk * 4 + u) * L
                            cv = plsc.load_gather(
                                row_v, [lab_v[pl.ds(off, L)]])
                            d = xv[pl.ds((blk * 4 + u) * L, L)] - cv
                            out.append(aa[u] + d * d)
                        return tuple(out)

                    a = lax.fori_loop(0, CHUNK // (4 * L), body, a)

                    @pl.when(p + 1 < n_pairs)
                    def _():
                        fire(fi, 2 * p + 2 + buf, buf)
                return a

            accs = lax.fori_loop(0, n_pairs, pair_body,
                                 (acc, zero, zero, zero))
            acc = (accs[0] + accs[1]) + (accs[2] + accs[3])

        for rr in range(8):
            for h in range(128 // L):
                ob_v[rr, pl.ds(h * L, L)] = jnp.zeros((L,), jnp.float32)
        ob_v[0, pl.ds(0, L)] = acc * inv_n
        pltpu.sync_copy(ob_v, out_hbm.at[wid])

    return sc_kernel


def kernel(x, labels, centers):
    B, D = x.shape
    V = centers.shape[0]
    info = plsc.get_sparse_core_info()
    NC, NS, L = info.num_cores, info.num_subcores, info.num_lanes
    NW = NC * NS

    sc_kernel = _make_sc_kernel(B, D, V, NC, NW, L)
    row_main = (V // 128) * 128
    tail = jnp.pad(centers[row_main:].T, ((0, 0), (0, 128 - (V - row_main))))
    partials = sc_kernel(
        x.T.reshape(D // 8, 8, B),
        labels.astype(jnp.int32),
        centers.T.reshape(D // 8, 8, V),
        tail.reshape(D // 8, 8, 128),
    )
    return jnp.sum(partials)


# speedup vs baseline: 1.2276x; 1.0478x over previous
import functools

import jax
import jax.numpy as jnp
from jax import lax
from jax.experimental import pallas as pl
from jax.experimental.pallas import tpu as pltpu
from jax.experimental.pallas import tpu_sc as plsc


def _make_sc_kernel(B, D, V, NC, NW, L):
    ROW_MAIN = (V // 128) * 128
    TAIL = V - ROW_MAIN
    F_PER_W = D // NW
    CHUNK = 4096
    n_chunks = B // CHUNK
    inv_n = 1.0 / (B * D)

    mesh = plsc.VectorSubcoreMesh(core_axis_name="c", subcore_axis_name="s")

    @functools.partial(
        pl.kernel,
        mesh=mesh,
        compiler_params=pltpu.CompilerParams(use_tc_tiling_on_sc=True,
                                             needs_layout_passes=False),
        out_type=jax.ShapeDtypeStruct((NW, 8, 128), jnp.float32),
        scratch_types=[
            pltpu.VMEM((ROW_MAIN + 128,), jnp.float32),
            pltpu.VMEM((B,), jnp.int32),
            pltpu.VMEM((CHUNK,), jnp.float32),
            pltpu.VMEM((CHUNK,), jnp.float32),
            pltpu.VMEM((8, 128), jnp.float32),
            pltpu.SemaphoreType.DMA,
            pltpu.SemaphoreType.DMA,
        ],
    )
    def sc_kernel(x_hbm, idx_hbm, tab_hbm, tail_hbm, out_hbm, row_v, lab_v,
                  xv0, xv1, ob_v, lsem, xsem):
        wid = lax.axis_index("s") * NC + lax.axis_index("c")
        xvs = (xv0, xv1)
        zero = jnp.zeros((L,), jnp.float32)
        acc = zero

        pltpu.async_copy(idx_hbm.at[pl.ds(0, B)], lab_v, lsem)

        fs, ts, rs = [], [], []
        for fi in range(F_PER_W):
            f = F_PER_W * wid + fi
            fs.append(f)
            ts.append(lax.shift_right_logical(f, 3))
            rs.append(lax.bitwise_and(f, 7))

        def fire(fi, c, buf):
            pltpu.async_copy(
                x_hbm.at[ts[fi], rs[fi], pl.ds(c * CHUNK, CHUNK)],
                xvs[buf], xsem)

        n_pairs = n_chunks // 2

        for fi in range(F_PER_W):
            t, r = ts[fi], rs[fi]
            fire(fi, 0, 0)
            fire(fi, 1, 1)
            pltpu.sync_copy(tab_hbm.at[t, r, pl.ds(0, ROW_MAIN)],
                            row_v.at[pl.ds(0, ROW_MAIN)])
            if TAIL:
                pltpu.sync_copy(tail_hbm.at[t, r, pl.ds(0, 128)],
                                row_v.at[pl.ds(ROW_MAIN, 128)])
            if fi == 0:
                pltpu.make_async_copy(idx_hbm.at[pl.ds(0, B)], lab_v,
                                      lsem).wait()

            def pair_body(p, a):
                for buf in range(2):
                    pltpu.make_async_copy(
                        x_hbm.at[t, r, pl.ds(0, CHUNK)], xvs[buf],
                        xsem).wait()
                    base = (2 * p + buf) * CHUNK
                    xv = xvs[buf]

                    def body(blk, aa):
                        out = []
                        for u in range(4):
                            off = base + (blk * 4 + u) * L
                            cv = plsc.load_gather(
                                row_v, [lab_v[pl.ds(off, L)]])
                            d = xv[pl.ds((blk * 4 + u) * L, L)] - cv
                            out.append(aa[u] + d * d)
                        return tuple(out)

                    a = lax.fori_loop(0, CHUNK // (4 * L), body, a)

                    @pl.when(p + 1 < n_pairs)
                    def _():
                        fire(fi, 2 * p + 2 + buf, buf)
                return a

            accs = lax.fori_loop(0, n_pairs, pair_body,
                                 (acc, zero, zero, zero))
            acc = (accs[0] + accs[1]) + (accs[2] + accs[3])

        for rr in range(8):
            for h in range(128 // L):
                ob_v[rr, pl.ds(h * L, L)] = jnp.zeros((L,), jnp.float32)
        ob_v[0, pl.ds(0, L)] = acc * inv_n
        pltpu.sync_copy(ob_v, out_hbm.at[wid])

    return sc_kernel


def kernel(x, labels, centers):
    B, D = x.shape
    V = centers.shape[0]
    info = plsc.get_sparse_core_info()
    NC, NS, L = info.num_cores, info.num_subcores, info.num_lanes
    NW = NC * NS

    sc_kernel = _make_sc_kernel(B, D, V, NC, NW, L)
    row_main = (V // 128) * 128
    tail = jnp.pad(centers[row_main:].T, ((0, 0), (0, 128 - (V - row_main))))
    partials = sc_kernel(
        x.T.reshape(D // 8, 8, B),
        labels.astype(jnp.int32),
        centers.T.reshape(D // 8, 8, V),
        tail.reshape(D // 8, 8, 128),
    )
    return jnp.sum(partials)
